# baseline (speedup 1.0000x reference)
import functools

import jax
import jax.numpy as jnp
from jax import lax
from jax.experimental import pallas as pl
from jax.experimental.pallas import tpu as pltpu
from jax.experimental.pallas import tpu_sc as plsc

N = 10000
E = 320000
D_IN = 128
D_H = 128
D_OUT = 64

NC = 2
NS = 16
NW = NC * NS
K = 128
NB0 = 80
NB1 = 80
NBMAX = max(NB0, NB1)
E_PAD = NS * K * (NB0 + NB1)
NP = 10240
BR = 128
CHUNK = NP // NS


def _deg_kernel(src_hbm, dst_hbm, zeros_hbm, ones_hbm, out_hbm,
                srci, dsti, ones_v, acc_out, acc_in, sem):
    cid = lax.axis_index("c")
    sid = lax.axis_index("s")
    widx = cid * NS + sid
    sl = pl.ds(sid * CHUNK, CHUNK)
    pltpu.sync_copy(zeros_hbm, acc_out.at[sl])
    pltpu.sync_copy(zeros_hbm, acc_in.at[sl])
    plsc.subcore_barrier()
    pltpu.sync_copy(ones_hbm, ones_v)
    pltpu.sync_copy(src_hbm.at[widx], srci)
    pltpu.sync_copy(dst_hbm.at[widx], dsti)

    def body(b, carry):
        pltpu.sync_copy(ones_v, acc_out.at[srci.at[b]], add=True)
        pltpu.sync_copy(ones_v, acc_in.at[dsti.at[b]], add=True)
        return carry

    nb = lax.select(cid == 0, NB0, NB1)
    lax.fori_loop(0, nb, body, 0)
    plsc.subcore_barrier()
    pltpu.sync_copy(acc_out.at[sl], out_hbm.at[cid, 0, sl])
    pltpu.sync_copy(acc_in.at[sl], out_hbm.at[cid, 1, sl])


CW = 16
NCH0 = NB0 // CW
NCH1 = NB1 // CW


def _make_agg(D):
    def agg_kernel(z_hbm, src_hbm, dst_hbm, w_hbm, zrows_hbm, out_hbm,
                   srcb, dstb, wb, r0, r1, g0, g1, s0, s1, ix0, ix1, acc):
        R = (r0, r1)
        G = (g0, g1)
        S = (s0, s1)
        IX = (ix0, ix1)
        cid = lax.axis_index("c")
        sid = lax.axis_index("s")
        widx = cid * NS + sid

        with jax.named_scope("agg_zero"):
            pltpu.sync_copy(zrows_hbm, r0)
            for i in range(CHUNK // K):
                pltpu.sync_copy(r0, acc.at[pl.ds(sid * CHUNK + i * K, K)])
            plsc.subcore_barrier()

        def start_stage(c, h):
            hs = pl.ds(lax.rem(c, 2) * CW, CW)
            pltpu.async_copy(src_hbm.at[widx, pl.ds(c * CW, CW)],
                             srcb.at[hs], IX[h])
            pltpu.async_copy(dst_hbm.at[widx, pl.ds(c * CW, CW)],
                             dstb.at[hs], IX[h])
            pltpu.async_copy(w_hbm.at[widx, pl.ds(c * CW * K, CW * K)],
                             wb.at[pl.ds(lax.rem(c, 2) * CW * K, CW * K)],
                             IX[h])

        def wait_stage(h):
            sl = pl.ds(0, CW)
            pltpu.make_async_copy(src_hbm.at[widx, sl], srcb.at[sl],
                                  IX[h]).wait()
            pltpu.make_async_copy(dst_hbm.at[widx, sl], dstb.at[sl],
                                  IX[h]).wait()
            pltpu.make_async_copy(w_hbm.at[widx, pl.ds(0, CW * K)],
                                  wb.at[pl.ds(0, CW * K)], IX[h]).wait()

        def start_gather(b, j):
            pltpu.async_copy(z_hbm.at[srcb.at[lax.rem(b, 2 * CW)]],
                             R[j], G[j])

        def wait_gather(j):
            pltpu.make_async_copy(z_hbm.at[srcb.at[0]], R[j], G[j]).wait()

        def start_scatter(b, j):
            pltpu.async_copy(R[j], acc.at[dstb.at[lax.rem(b, 2 * CW)]],
                             S[j], add=True)

        def wait_scatter(j):
            pltpu.make_async_copy(R[j], acc.at[dstb.at[0]], S[j]).wait()

        dnums = lax.GatherDimensionNumbers(
            offset_dims=(), collapsed_slice_dims=(0,), start_index_map=(0,))

        def mult(b, j):
            wbase = lax.rem(b, 2 * CW) * K

            def gbody(g, carry):
                w16 = wb[pl.ds(wbase + g * 16, 16)]
                e0 = g * 16
                for k_ in range(16):
                    e = e0 + k_
                    wspl = lax.gather(
                        w16, jnp.full((16, 1), k_, jnp.int32), dnums, (1,),
                        mode=lax.GatherScatterMode.PROMISE_IN_BOUNDS)
                    for jj in range(D // 16):
                        sl = pl.ds(jj * 16, 16)
                        R[j][e, sl] = R[j][e, sl] * wspl
                return carry
            lax.fori_loop(0, K // 16, gbody, 0)

        nb = lax.select(cid == 0, NB0, NB1)
        nch = lax.select(cid == 0, NCH0, NCH1)

        start_stage(0, 0)
        start_stage(1, 1)
        wait_stage(0)
        start_gather(0, 0)
        start_gather(1, 1)

        def chunk_body(c, carry):
            def pair(p, carry2):
                b0 = c * CW + 2 * p
                b1 = b0 + 1

                @pl.when((p == 0) & (c >= 1) & (c + 1 < nch))
                def _():
                    @pl.when(lax.rem(c + 1, 2) == 0)
                    def _():
                        start_stage(c + 1, 0)

                    @pl.when(lax.rem(c + 1, 2) == 1)
                    def _():
                        start_stage(c + 1, 1)

                @pl.when((p == CW // 2 - 2) & (c + 1 < nch))
                def _():
                    @pl.when(lax.rem(c + 1, 2) == 0)
                    def _():
                        wait_stage(0)

                    @pl.when(lax.rem(c + 1, 2) == 1)
                    def _():
                        wait_stage(1)

                wait_gather(0)
                mult(b0, 0)
                start_scatter(b0, 0)
                wait_gather(1)
                mult(b1, 1)
                start_scatter(b1, 1)
                wait_scatter(0)
                start_gather(jnp.minimum(b0 + 2, nb - 1), 0)
                wait_scatter(1)
                start_gather(jnp.minimum(b1 + 2, nb - 1), 1)
                return carry2

            lax.fori_loop(0, CW // 2, pair, carry)
            return carry

        with jax.named_scope("agg_loop"):
            lax.fori_loop(0, nch, chunk_body, 0)
            wait_gather(0)
            wait_gather(1)
        with jax.named_scope("agg_tail"):
            plsc.subcore_barrier()
        sl = pl.ds(sid * CHUNK, CHUNK)
        pltpu.sync_copy(acc.at[sl], out_hbm.at[cid, sl])

    return agg_kernel


def _tc1_body(degp, x, w, z, ns, nd):
    d_out = degp[0, 0, :] + degp[1, 0, :]
    d_in = degp[0, 1, :] + degp[1, 1, :]
    ns_v = lax.rsqrt(jnp.maximum(d_out, 1.0))
    nd_v = lax.rsqrt(jnp.maximum(d_in, 1.0))
    ns[...] = ns_v
    nd[...] = nd_v
    z[...] = jnp.dot(x[...], w[...],
                     preferred_element_type=jnp.float32) * ns_v[:, None]


def _tcpost_body(p, nd, b, w, ns, z):
    agg = (p[0] + p[1]) * nd[...][:, None]
    h = jnp.maximum(agg + b[...][None, :], 0.0)
    z[...] = jnp.dot(h, w[...],
                     preferred_element_type=jnp.float32) * ns[...][:, None]


def _tcfin_body(p, nd, b, out):
    agg = p[0, :, :D_OUT] + p[1, :, :D_OUT]
    out[...] = agg * nd[...][:, None] + b[...][None, :]


def kernel(x, edge_index, edge_weight, W1, b1, W2, b2, W3, b3):
    f32 = jnp.float32
    src = edge_index[0]
    dst = edge_index[1]
    padn = E_PAD - E

    def layout(arr, fill):
        e0 = NS * NB0 * K
        p0 = arr[:e0].reshape(NS, NB0, K)
        p1 = arr[e0:].reshape(NS, NB1, K)
        p0 = jnp.pad(p0, ((0, 0), (0, NBMAX - NB0), (0, 0)),
                     constant_values=fill)
        p1 = jnp.pad(p1, ((0, 0), (0, NBMAX - NB1), (0, 0)),
                     constant_values=fill)
        return jnp.concatenate([p0, p1], axis=0)

    trash = (N + jnp.arange(padn, dtype=jnp.int32) % (NP - N)).astype(
        jnp.int32)
    src_r = layout(jnp.concatenate([src, trash]), N)
    dst_r = layout(jnp.concatenate([dst, trash]), N)
    w_r = layout(jnp.concatenate(
        [edge_weight, jnp.zeros((padn,), f32)]), 0).reshape(NW, NBMAX * K)
    x_pad = jnp.zeros((NP, D_IN), f32).at[:N].set(x)

    zrows128 = jnp.zeros((K, D_H), f32)
    W3p = jnp.zeros((D_H, D_H), f32).at[:, :D_OUT].set(W3)

    deg_call = pl.kernel(
        _deg_kernel,
        out_type=jax.ShapeDtypeStruct((NC, 2, NP), f32),
        mesh=plsc.VectorSubcoreMesh(core_axis_name="c", subcore_axis_name="s"),
        scratch_types=[
            pltpu.VMEM((NBMAX, K), jnp.int32),
            pltpu.VMEM((NBMAX, K), jnp.int32),
            pltpu.VMEM((K,), f32),
            pltpu.VMEM_SHARED((NP,), f32),
            pltpu.VMEM_SHARED((NP,), f32),
            pltpu.SemaphoreType.DMA,
        ],
    )
    degp = deg_call(src_r, dst_r, jnp.zeros((CHUNK,), f32),
                    jnp.ones((K,), f32))

    grid = (NP // BR,)
    z1, ns, nd = pl.pallas_call(
        _tc1_body,
        grid=grid,
        in_specs=[
            pl.BlockSpec((NC, 2, BR), lambda i: (0, 0, i)),
            pl.BlockSpec((BR, D_IN), lambda i: (i, 0)),
            pl.BlockSpec((D_IN, D_H), lambda i: (0, 0)),
        ],
        out_specs=[
            pl.BlockSpec((BR, D_H), lambda i: (i, 0)),
            pl.BlockSpec((BR,), lambda i: (i,)),
            pl.BlockSpec((BR,), lambda i: (i,)),
        ],
        out_shape=[
            jax.ShapeDtypeStruct((NP, D_H), f32),
            jax.ShapeDtypeStruct((NP,), f32),
            jax.ShapeDtypeStruct((NP,), f32),
        ],
    )(degp, x_pad, W1)

    def make_agg_call(D):
        return pl.kernel(
            _make_agg(D),
            out_type=jax.ShapeDtypeStruct((NC, NP, D), f32),
            mesh=plsc.VectorSubcoreMesh(core_axis_name="c",
                                        subcore_axis_name="s"),
            scratch_types=(
                [pltpu.VMEM((2 * CW, K), jnp.int32),
                 pltpu.VMEM((2 * CW, K), jnp.int32),
                 pltpu.VMEM((2 * CW * K,), f32)]
                + [pltpu.VMEM((K, D), f32)] * 2
                + [pltpu.SemaphoreType.DMA] * 6
                + [pltpu.VMEM_SHARED((NP, D), f32)]
            ),
        )
    agg128 = make_agg_call(D_H)

    def tcpost_call(p, ndv, bvec, W, nsv, Dp):
        Din = W.shape[0]
        return pl.pallas_call(
            _tcpost_body,
            grid=grid,
            in_specs=[
                pl.BlockSpec((NC, BR, Din), lambda i: (0, i, 0)),
                pl.BlockSpec((BR,), lambda i: (i,)),
                pl.BlockSpec((Din,), lambda i: (0,)),
                pl.BlockSpec((Din, Dp), lambda i: (0, 0)),
                pl.BlockSpec((BR,), lambda i: (i,)),
            ],
            out_specs=pl.BlockSpec((BR, Dp), lambda i: (i, 0)),
            out_shape=jax.ShapeDtypeStruct((NP, Dp), f32),
        )(p, ndv, bvec, W, nsv)

    p1 = agg128(z1, src_r, dst_r, w_r, zrows128)
    z2 = tcpost_call(p1, nd, b1, W2, ns, D_H)
    p2 = agg128(z2, src_r, dst_r, w_r, zrows128)
    z3 = tcpost_call(p2, nd, b2, W3p, ns, D_H)
    p3 = agg128(z3, src_r, dst_r, w_r, zrows128)
    out_pad = pl.pallas_call(
        _tcfin_body,
        grid=grid,
        in_specs=[
            pl.BlockSpec((NC, BR, D_H), lambda i: (0, i, 0)),
            pl.BlockSpec((BR,), lambda i: (i,)),
            pl.BlockSpec((D_OUT,), lambda i: (0,)),
        ],
        out_specs=pl.BlockSpec((BR, D_OUT), lambda i: (i, 0)),
        out_shape=jax.ShapeDtypeStruct((NP, D_OUT), f32),
    )(p3, nd, b3)
    return out_pad[:N]


# speedup vs baseline: 1.0881x; 1.0881x over previous
import functools

import jax
import jax.numpy as jnp
from jax import lax
from jax.experimental import pallas as pl
from jax.experimental.pallas import tpu as pltpu
from jax.experimental.pallas import tpu_sc as plsc

N = 10000
E = 320000
D_IN = 128
D_H = 128
D_OUT = 64

NC = 2
NS = 16
NW = NC * NS
K = 128
NB0 = 80
NB1 = 80
NBMAX = max(NB0, NB1)
E_PAD = NS * K * (NB0 + NB1)
NP = 10240
BR = 128
CHUNK = NP // NS


def _deg_kernel(src_hbm, dst_hbm, zeros_hbm, ones_hbm, out_hbm,
                srci, dsti, ones_v, acc_out, acc_in, sem):
    cid = lax.axis_index("c")
    sid = lax.axis_index("s")
    widx = cid * NS + sid
    sl = pl.ds(sid * CHUNK, CHUNK)
    pltpu.sync_copy(zeros_hbm, acc_out.at[sl])
    pltpu.sync_copy(zeros_hbm, acc_in.at[sl])
    plsc.subcore_barrier()
    pltpu.sync_copy(ones_hbm, ones_v)
    pltpu.sync_copy(src_hbm.at[widx], srci)
    pltpu.sync_copy(dst_hbm.at[widx], dsti)

    def body(b, carry):
        pltpu.sync_copy(ones_v, acc_out.at[srci.at[b]], add=True)
        pltpu.sync_copy(ones_v, acc_in.at[dsti.at[b]], add=True)
        return carry

    nb = lax.select(cid == 0, NB0, NB1)
    lax.fori_loop(0, nb, body, 0)
    plsc.subcore_barrier()
    pltpu.sync_copy(acc_out.at[sl], out_hbm.at[cid, 0, sl])
    pltpu.sync_copy(acc_in.at[sl], out_hbm.at[cid, 1, sl])


CW = 16
NCH0 = NB0 // CW
NCH1 = NB1 // CW


def _make_agg(D):
    def agg_kernel(z_hbm, src_hbm, dst_hbm, w_hbm, zrows_hbm, out_hbm,
                   srcb, dstb, wb, r0, r1, g0, g1, s0, s1, ix0, ix1, acc):
        R = (r0, r1)
        G = (g0, g1)
        S = (s0, s1)
        IX = (ix0, ix1)
        cid = lax.axis_index("c")
        sid = lax.axis_index("s")
        widx = cid * NS + sid

        with jax.named_scope("agg_zero"):
            pltpu.sync_copy(zrows_hbm, r0)
            for i in range(CHUNK // K):
                pltpu.sync_copy(r0, acc.at[pl.ds(sid * CHUNK + i * K, K)])
            plsc.subcore_barrier()

        def start_stage(c, h):
            hs = pl.ds(lax.rem(c, 2) * CW, CW)
            pltpu.async_copy(src_hbm.at[widx, pl.ds(c * CW, CW)],
                             srcb.at[hs], IX[h])
            pltpu.async_copy(dst_hbm.at[widx, pl.ds(c * CW, CW)],
                             dstb.at[hs], IX[h])
            pltpu.async_copy(w_hbm.at[widx, pl.ds(c * CW * K, CW * K)],
                             wb.at[pl.ds(lax.rem(c, 2) * CW * K, CW * K)],
                             IX[h])

        def wait_stage(h):
            sl = pl.ds(0, CW)
            pltpu.make_async_copy(src_hbm.at[widx, sl], srcb.at[sl],
                                  IX[h]).wait()
            pltpu.make_async_copy(dst_hbm.at[widx, sl], dstb.at[sl],
                                  IX[h]).wait()
            pltpu.make_async_copy(w_hbm.at[widx, pl.ds(0, CW * K)],
                                  wb.at[pl.ds(0, CW * K)], IX[h]).wait()

        def start_gather(b, j):
            pltpu.async_copy(z_hbm.at[srcb.at[lax.rem(b, 2 * CW)]],
                             R[j], G[j])

        def wait_gather(j):
            pltpu.make_async_copy(z_hbm.at[srcb.at[0]], R[j], G[j]).wait()

        def start_scatter(b, j):
            pltpu.async_copy(R[j], acc.at[dstb.at[lax.rem(b, 2 * CW)]],
                             S[j], add=True)

        def wait_scatter(j):
            pltpu.make_async_copy(R[j], acc.at[dstb.at[0]], S[j]).wait()

        dnums = lax.GatherDimensionNumbers(
            offset_dims=(), collapsed_slice_dims=(0,), start_index_map=(0,))

        def mult(b, j):
            wbase = lax.rem(b, 2 * CW) * K

            def gbody(g, carry):
                w16 = wb[pl.ds(wbase + g * 16, 16)]
                e0 = g * 16
                for k_ in range(16):
                    e = e0 + k_
                    wspl = lax.gather(
                        w16, jnp.full((16, 1), k_, jnp.int32), dnums, (1,),
                        mode=lax.GatherScatterMode.PROMISE_IN_BOUNDS)
                    for jj in range(D // 16):
                        sl = pl.ds(jj * 16, 16)
                        R[j][e, sl] = R[j][e, sl] * wspl
                return carry
            lax.fori_loop(0, K // 16, gbody, 0)

        nb = lax.select(cid == 0, NB0, NB1)
        nch = lax.select(cid == 0, NCH0, NCH1)

        start_stage(0, 0)
        start_stage(1, 1)
        wait_stage(0)
        start_gather(0, 0)
        start_gather(1, 1)

        def chunk_body(c, carry):
            def pair(p, carry2):
                b0 = c * CW + 2 * p
                b1 = b0 + 1

                @pl.when((p == 0) & (c >= 1) & (c + 1 < nch))
                def _():
                    @pl.when(lax.rem(c + 1, 2) == 0)
                    def _():
                        start_stage(c + 1, 0)

                    @pl.when(lax.rem(c + 1, 2) == 1)
                    def _():
                        start_stage(c + 1, 1)

                @pl.when((p == CW // 2 - 2) & (c + 1 < nch))
                def _():
                    @pl.when(lax.rem(c + 1, 2) == 0)
                    def _():
                        wait_stage(0)

                    @pl.when(lax.rem(c + 1, 2) == 1)
                    def _():
                        wait_stage(1)

                wait_gather(0)
                mult(b0, 0)
                start_scatter(b0, 0)
                wait_scatter(0)
                start_gather(jnp.minimum(b0 + 2, nb - 1), 0)
                wait_gather(1)
                mult(b1, 1)
                start_scatter(b1, 1)
                wait_scatter(1)
                start_gather(jnp.minimum(b1 + 2, nb - 1), 1)
                return carry2

            lax.fori_loop(0, CW // 2, pair, carry)
            return carry

        with jax.named_scope("agg_loop"):
            lax.fori_loop(0, nch, chunk_body, 0)
            wait_gather(0)
            wait_gather(1)
        with jax.named_scope("agg_tail"):
            plsc.subcore_barrier()
        sl = pl.ds(sid * CHUNK, CHUNK)
        pltpu.sync_copy(acc.at[sl], out_hbm.at[cid, sl])

    return agg_kernel


def _tc1_body(degp, x, w, z, ns, nd):
    d_out = degp[0, 0, :] + degp[1, 0, :]
    d_in = degp[0, 1, :] + degp[1, 1, :]
    ns_v = lax.rsqrt(jnp.maximum(d_out, 1.0))
    nd_v = lax.rsqrt(jnp.maximum(d_in, 1.0))
    ns[...] = ns_v
    nd[...] = nd_v
    z[...] = jnp.dot(x[...], w[...],
                     preferred_element_type=jnp.float32) * ns_v[:, None]


def _tcpost_body(p, nd, b, w, ns, z):
    agg = (p[0] + p[1]) * nd[...][:, None]
    h = jnp.maximum(agg + b[...][None, :], 0.0)
    z[...] = jnp.dot(h, w[...],
                     preferred_element_type=jnp.float32) * ns[...][:, None]


def _tcfin_body(p, nd, b, out):
    agg = p[0, :, :D_OUT] + p[1, :, :D_OUT]
    out[...] = agg * nd[...][:, None] + b[...][None, :]


def kernel(x, edge_index, edge_weight, W1, b1, W2, b2, W3, b3):
    f32 = jnp.float32
    src = edge_index[0]
    dst = edge_index[1]
    padn = E_PAD - E

    def layout(arr, fill):
        e0 = NS * NB0 * K
        p0 = arr[:e0].reshape(NS, NB0, K)
        p1 = arr[e0:].reshape(NS, NB1, K)
        p0 = jnp.pad(p0, ((0, 0), (0, NBMAX - NB0), (0, 0)),
                     constant_values=fill)
        p1 = jnp.pad(p1, ((0, 0), (0, NBMAX - NB1), (0, 0)),
                     constant_values=fill)
        return jnp.concatenate([p0, p1], axis=0)

    trash = (N + jnp.arange(padn, dtype=jnp.int32) % (NP - N)).astype(
        jnp.int32)
    src_r = layout(jnp.concatenate([src, trash]), N)
    dst_r = layout(jnp.concatenate([dst, trash]), N)
    w_r = layout(jnp.concatenate(
        [edge_weight, jnp.zeros((padn,), f32)]), 0).reshape(NW, NBMAX * K)
    x_pad = jnp.zeros((NP, D_IN), f32).at[:N].set(x)

    zrows128 = jnp.zeros((K, D_H), f32)
    W3p = jnp.zeros((D_H, D_H), f32).at[:, :D_OUT].set(W3)

    deg_call = pl.kernel(
        _deg_kernel,
        out_type=jax.ShapeDtypeStruct((NC, 2, NP), f32),
        mesh=plsc.VectorSubcoreMesh(core_axis_name="c", subcore_axis_name="s"),
        scratch_types=[
            pltpu.VMEM((NBMAX, K), jnp.int32),
            pltpu.VMEM((NBMAX, K), jnp.int32),
            pltpu.VMEM((K,), f32),
            pltpu.VMEM_SHARED((NP,), f32),
            pltpu.VMEM_SHARED((NP,), f32),
            pltpu.SemaphoreType.DMA,
        ],
    )
    degp = deg_call(src_r, dst_r, jnp.zeros((CHUNK,), f32),
                    jnp.ones((K,), f32))

    grid = (NP // BR,)
    z1, ns, nd = pl.pallas_call(
        _tc1_body,
        grid=grid,
        in_specs=[
            pl.BlockSpec((NC, 2, BR), lambda i: (0, 0, i)),
            pl.BlockSpec((BR, D_IN), lambda i: (i, 0)),
            pl.BlockSpec((D_IN, D_H), lambda i: (0, 0)),
        ],
        out_specs=[
            pl.BlockSpec((BR, D_H), lambda i: (i, 0)),
            pl.BlockSpec((BR,), lambda i: (i,)),
            pl.BlockSpec((BR,), lambda i: (i,)),
        ],
        out_shape=[
            jax.ShapeDtypeStruct((NP, D_H), f32),
            jax.ShapeDtypeStruct((NP,), f32),
            jax.ShapeDtypeStruct((NP,), f32),
        ],
    )(degp, x_pad, W1)

    def make_agg_call(D):
        return pl.kernel(
            _make_agg(D),
            out_type=jax.ShapeDtypeStruct((NC, NP, D), f32),
            mesh=plsc.VectorSubcoreMesh(core_axis_name="c",
                                        subcore_axis_name="s"),
            scratch_types=(
                [pltpu.VMEM((2 * CW, K), jnp.int32),
                 pltpu.VMEM((2 * CW, K), jnp.int32),
                 pltpu.VMEM((2 * CW * K,), f32)]
                + [pltpu.VMEM((K, D), f32)] * 2
                + [pltpu.SemaphoreType.DMA] * 6
                + [pltpu.VMEM_SHARED((NP, D), f32)]
            ),
        )
    agg128 = make_agg_call(D_H)

    def tcpost_call(p, ndv, bvec, W, nsv, Dp):
        Din = W.shape[0]
        return pl.pallas_call(
            _tcpost_body,
            grid=grid,
            in_specs=[
                pl.BlockSpec((NC, BR, Din), lambda i: (0, i, 0)),
                pl.BlockSpec((BR,), lambda i: (i,)),
                pl.BlockSpec((Din,), lambda i: (0,)),
                pl.BlockSpec((Din, Dp), lambda i: (0, 0)),
                pl.BlockSpec((BR,), lambda i: (i,)),
            ],
            out_specs=pl.BlockSpec((BR, Dp), lambda i: (i, 0)),
            out_shape=jax.ShapeDtypeStruct((NP, Dp), f32),
        )(p, ndv, bvec, W, nsv)

    p1 = agg128(z1, src_r, dst_r, w_r, zrows128)
    z2 = tcpost_call(p1, nd, b1, W2, ns, D_H)
    p2 = agg128(z2, src_r, dst_r, w_r, zrows128)
    z3 = tcpost_call(p2, nd, b2, W3p, ns, D_H)
    p3 = agg128(z3, src_r, dst_r, w_r, zrows128)
    out_pad = pl.pallas_call(
        _tcfin_body,
        grid=grid,
        in_specs=[
            pl.BlockSpec((NC, BR, D_H), lambda i: (0, i, 0)),
            pl.BlockSpec((BR,), lambda i: (i,)),
            pl.BlockSpec((D_OUT,), lambda i: (0,)),
        ],
        out_specs=pl.BlockSpec((BR, D_OUT), lambda i: (i, 0)),
        out_shape=jax.ShapeDtypeStruct((NP, D_OUT), f32),
    )(p3, nd, b3)
    return out_pad[:N]
